# baseline (speedup 1.0000x reference)
import jax
import jax.numpy as jnp
from jax.experimental import pallas as pl
from jax.experimental.pallas import tpu as pltpu

_B, _C, _H, _W = 64, 384, 14, 14
_E = 8
_K = 2
_HID = 2 * _C
_HW = _H * _W
_SPB = 16


def _fused_kernel(x_ref, wrt_ref, br_ref, ws_ref, gamma_ref, beta_ref,
                  w1_ref, w2_ref, out_ref):
    xb2 = x_ref[...]
    pooled = jnp.mean(xb2, axis=2)
    logits = jax.lax.dot_general(
        pooled, wrt_ref[...], (((1,), (0,)), ((), ())),
        preferred_element_type=jnp.float32) + br_ref[...]
    probs = jax.nn.softmax(logits, axis=-1)
    lane = jax.lax.broadcasted_iota(jnp.int32, probs.shape, 1)
    a1 = jnp.argmax(probs, axis=-1)
    masked = jnp.where(lane == a1[:, None], -jnp.inf, probs)

    xb16_all = xb2.astype(jnp.bfloat16)
    for r in range(_SPB):
        xb = xb2[r]
        xb16 = xb16_all[r]
        pr = probs[r]
        mr = masked[r]
        e0 = a1[r]
        m1 = jnp.max(pr)
        e1 = jnp.argmax(mr)
        m2 = jnp.max(mr)
        denom = m1 + m2
        w0 = m1 / denom
        w1 = m2 / denom

        h0 = jnp.dot(w1_ref[e0], xb16, preferred_element_type=jnp.float32)
        h0 = h0 * jax.nn.sigmoid(h0)
        out0 = jnp.dot(w2_ref[e0], h0.astype(jnp.bfloat16),
                       preferred_element_type=jnp.float32)

        h1 = jnp.dot(w1_ref[e1], xb16, preferred_element_type=jnp.float32)
        h1 = h1 * jax.nn.sigmoid(h1)
        out1 = jnp.dot(w2_ref[e1], h1.astype(jnp.bfloat16),
                       preferred_element_type=jnp.float32)

        shared = jnp.dot(ws_ref[...], xb16, preferred_element_type=jnp.float32)
        shared = shared * gamma_ref[...] + beta_ref[...]
        shared = shared * jax.nn.sigmoid(shared)

        out_ref[r] = xb + shared + w0 * out0 + w1 * out1


def kernel(x, Wr, br, Ws, gamma, beta, W1, W2):
    xr = x.reshape(_B, _C, _HW)
    out = pl.pallas_call(
        _fused_kernel,
        grid=(_B // _SPB,),
        in_specs=[
            pl.BlockSpec((_SPB, _C, _HW), lambda s: (s, 0, 0)),
            pl.BlockSpec((_C, _E), lambda s: (0, 0)),
            pl.BlockSpec((1, _E), lambda s: (0, 0)),
            pl.BlockSpec((_C, _C), lambda s: (0, 0)),
            pl.BlockSpec((_C, 1), lambda s: (0, 0)),
            pl.BlockSpec((_C, 1), lambda s: (0, 0)),
            pl.BlockSpec((_E, _HID, _C), lambda s: (0, 0, 0)),
            pl.BlockSpec((_E, _C, _HID), lambda s: (0, 0, 0)),
        ],
        out_specs=pl.BlockSpec((_SPB, _C, _HW), lambda s: (s, 0, 0)),
        out_shape=jax.ShapeDtypeStruct((_B, _C, _HW), jnp.float32),
        compiler_params=pltpu.CompilerParams(
            dimension_semantics=("arbitrary",)),
    )(xr, Wr.T, br.reshape(1, _E), Ws.astype(jnp.bfloat16),
      gamma.reshape(_C, 1), beta.reshape(_C, 1),
      W1.astype(jnp.bfloat16), W2.astype(jnp.bfloat16))
    return out.reshape(_B, _C, _H, _W)


# speedup vs baseline: 1.0738x; 1.0738x over previous
import jax
import jax.numpy as jnp
from jax.experimental import pallas as pl
from jax.experimental.pallas import tpu as pltpu


def _sigmoid(v):
    return 0.5 * jnp.tanh(0.5 * v) + 0.5

_B, _C, _H, _W = 64, 384, 14, 14
_E = 8
_K = 2
_HID = 2 * _C
_HW = _H * _W
_SPB = 16


def _fused_kernel(x_ref, wrt_ref, br_ref, ws_ref, gamma_ref, beta_ref,
                  w1_ref, w2_ref, out_ref):
    xb2 = x_ref[...]
    pooled = jnp.mean(xb2, axis=2)
    logits = jax.lax.dot_general(
        pooled, wrt_ref[...], (((1,), (0,)), ((), ())),
        preferred_element_type=jnp.float32) + br_ref[...]
    probs = jax.nn.softmax(logits, axis=-1)
    lane = jax.lax.broadcasted_iota(jnp.int32, probs.shape, 1)
    a1 = jnp.argmax(probs, axis=-1)
    masked = jnp.where(lane == a1[:, None], -jnp.inf, probs)

    for r in range(_SPB):
        xb = xb2[r]
        pr = probs[r]
        mr = masked[r]
        e0 = a1[r]
        m1 = jnp.max(pr)
        e1 = jnp.argmax(mr)
        m2 = jnp.max(mr)
        denom = m1 + m2
        w0 = m1 / denom
        w1 = m2 / denom

        h0 = jnp.dot(w1_ref[e0], xb, preferred_element_type=jnp.float32)
        h0 = h0 * _sigmoid(h0)
        out0 = jnp.dot(w2_ref[e0], h0, preferred_element_type=jnp.float32)

        h1 = jnp.dot(w1_ref[e1], xb, preferred_element_type=jnp.float32)
        h1 = h1 * _sigmoid(h1)
        out1 = jnp.dot(w2_ref[e1], h1, preferred_element_type=jnp.float32)

        shared = jnp.dot(ws_ref[...], xb, preferred_element_type=jnp.float32)
        shared = shared * gamma_ref[...] + beta_ref[...]
        shared = shared * _sigmoid(shared)

        out_ref[r] = xb + shared + w0 * out0 + w1 * out1


def kernel(x, Wr, br, Ws, gamma, beta, W1, W2):
    xr = x.reshape(_B, _C, _HW)
    out = pl.pallas_call(
        _fused_kernel,
        grid=(_B // _SPB,),
        in_specs=[
            pl.BlockSpec((_SPB, _C, _HW), lambda s: (s, 0, 0)),
            pl.BlockSpec((_C, _E), lambda s: (0, 0)),
            pl.BlockSpec((1, _E), lambda s: (0, 0)),
            pl.BlockSpec((_C, _C), lambda s: (0, 0)),
            pl.BlockSpec((_C, 1), lambda s: (0, 0)),
            pl.BlockSpec((_C, 1), lambda s: (0, 0)),
            pl.BlockSpec((_E, _HID, _C), lambda s: (0, 0, 0)),
            pl.BlockSpec((_E, _C, _HID), lambda s: (0, 0, 0)),
        ],
        out_specs=pl.BlockSpec((_SPB, _C, _HW), lambda s: (s, 0, 0)),
        out_shape=jax.ShapeDtypeStruct((_B, _C, _HW), jnp.float32),
        compiler_params=pltpu.CompilerParams(
            dimension_semantics=("arbitrary",)),
    )(xr, Wr.T, br.reshape(1, _E), Ws, gamma.reshape(_C, 1),
      beta.reshape(_C, 1), W1, W2)
    return out.reshape(_B, _C, _H, _W)


# speedup vs baseline: 1.0751x; 1.0012x over previous
import jax
import jax.numpy as jnp
from jax.experimental import pallas as pl
from jax.experimental.pallas import tpu as pltpu


def _sigmoid(v):
    return 0.5 * jnp.tanh(0.5 * v) + 0.5

_B, _C, _H, _W = 64, 384, 14, 14
_E = 8
_K = 2
_HID = 2 * _C
_HW = _H * _W
_SPB = 16


def _fused_kernel(x_ref, wrt_ref, br_ref, ws_ref, gamma_ref, beta_ref,
                  w1_ref, w2_ref, out_ref):
    with jax.named_scope("routing"):
        xb2 = x_ref[...]
        pooled = jnp.mean(xb2, axis=2)
        logits = jax.lax.dot_general(
            pooled, wrt_ref[...], (((1,), (0,)), ((), ())),
            preferred_element_type=jnp.float32) + br_ref[...]
        probs = jax.nn.softmax(logits, axis=-1)
        lane = jax.lax.broadcasted_iota(jnp.int32, probs.shape, 1)
        a1 = jnp.argmax(probs, axis=-1)
        masked = jnp.where(lane == a1[:, None], -jnp.inf, probs)

    for r in range(_SPB):
        xb = xb2[r]
        pr = probs[r]
        mr = masked[r]
        e0 = a1[r]
        m1 = jnp.max(pr)
        e1 = jnp.argmax(mr)
        m2 = jnp.max(mr)
        denom = m1 + m2
        w0 = m1 / denom
        w1 = m2 / denom

        with jax.named_scope("experts"):
            h0 = jnp.dot(w1_ref[e0], xb, preferred_element_type=jnp.float32)
            h0 = h0 * _sigmoid(h0)
            out0 = jnp.dot(w2_ref[e0], h0, preferred_element_type=jnp.float32)

            h1 = jnp.dot(w1_ref[e1], xb, preferred_element_type=jnp.float32)
            h1 = h1 * _sigmoid(h1)
            out1 = jnp.dot(w2_ref[e1], h1, preferred_element_type=jnp.float32)

        with jax.named_scope("shared"):
            shared = jnp.dot(ws_ref[...], xb, preferred_element_type=jnp.float32)
            shared = shared * gamma_ref[...] + beta_ref[...]
            shared = shared * _sigmoid(shared)

            out_ref[r] = xb + shared + w0 * out0 + w1 * out1


def kernel(x, Wr, br, Ws, gamma, beta, W1, W2):
    xr = x.reshape(_B, _C, _HW)
    out = pl.pallas_call(
        _fused_kernel,
        grid=(_B // _SPB,),
        in_specs=[
            pl.BlockSpec((_SPB, _C, _HW), lambda s: (s, 0, 0)),
            pl.BlockSpec((_C, _E), lambda s: (0, 0)),
            pl.BlockSpec((1, _E), lambda s: (0, 0)),
            pl.BlockSpec((_C, _C), lambda s: (0, 0)),
            pl.BlockSpec((_C, 1), lambda s: (0, 0)),
            pl.BlockSpec((_C, 1), lambda s: (0, 0)),
            pl.BlockSpec((_E, _HID, _C), lambda s: (0, 0, 0)),
            pl.BlockSpec((_E, _C, _HID), lambda s: (0, 0, 0)),
        ],
        out_specs=pl.BlockSpec((_SPB, _C, _HW), lambda s: (s, 0, 0)),
        out_shape=jax.ShapeDtypeStruct((_B, _C, _HW), jnp.float32),
        compiler_params=pltpu.CompilerParams(
            dimension_semantics=("arbitrary",)),
    )(xr, Wr.T, br.reshape(1, _E), Ws, gamma.reshape(_C, 1),
      beta.reshape(_C, 1), W1, W2)
    return out.reshape(_B, _C, _H, _W)
